# baseline (speedup 1.0000x reference)
import jax
import jax.numpy as jnp
from jax import lax
from jax.experimental import pallas as pl
from jax.experimental.pallas import tpu as pltpu
from jax.experimental.pallas import tpu_sc as plsc

B = 16384
C = 1000
NC = 2
NS = 16
NW = NC * NS
RPW = B // NW
GPW = RPW // 16
NBUF = 4


def _one_hot_body(tgt_hbm, out_hbm, idx_v, stage_v, sem):
    cid = lax.axis_index("c")
    sid = lax.axis_index("s")
    wid = sid * NC + cid
    base = pl.multiple_of(wid * RPW, 8)

    zeros16 = jnp.zeros((16,), jnp.float32)
    ones16 = jnp.ones((16,), jnp.float32)
    lanes = lax.iota(jnp.int32, 16)

    @pl.loop(0, NBUF)
    def _zs(s):
        @pl.loop(0, 16)
        def _zr(r):
            @pl.loop(0, 62)
            def _zc(i):
                stage_v[s, r, pl.ds(pl.multiple_of(i * 16, 16), 16)] = zeros16
            stage_v[s, r, pl.ds(984, 16)] = zeros16

    pltpu.sync_copy(tgt_hbm.at[pl.ds(base, RPW)], idx_v)

    @pl.loop(0, GPW)
    def _grp(g):
        slot = jnp.bitwise_and(g, NBUF - 1)

        @pl.when(g >= NBUF)
        def _retire():
            gp = g - NBUF
            pltpu.make_async_copy(
                stage_v.at[slot],
                out_hbm.at[pl.ds(pl.multiple_of(base + gp * 16, 8), 16)],
                sem.at[slot]).wait()
            cp16 = idx_v[pl.ds(pl.multiple_of(gp * 16, 16), 16)]
            plsc.store_scatter(stage_v.at[slot], [lanes, cp16], zeros16)

        c16 = idx_v[pl.ds(pl.multiple_of(g * 16, 16), 16)]
        plsc.store_scatter(stage_v.at[slot], [lanes, c16], ones16)
        pltpu.async_copy(
            stage_v.at[slot],
            out_hbm.at[pl.ds(pl.multiple_of(base + g * 16, 8), 16)],
            sem.at[slot])

    for t in range(NBUF):
        gp = GPW - NBUF + t
        pltpu.make_async_copy(
            stage_v.at[gp % NBUF],
            out_hbm.at[pl.ds(pl.multiple_of(base + gp * 16, 8), 16)],
            sem.at[gp % NBUF]).wait()


def kernel(target):
    mesh = plsc.VectorSubcoreMesh(core_axis_name="c", subcore_axis_name="s")
    f = pl.kernel(
        _one_hot_body,
        out_type=jax.ShapeDtypeStruct((B, C), jnp.float32),
        mesh=mesh,
        compiler_params=pltpu.CompilerParams(needs_layout_passes=False),
        scratch_types=[
            pltpu.VMEM((RPW,), jnp.int32),
            pltpu.VMEM((NBUF, 16, C), jnp.float32),
            pltpu.SemaphoreType.DMA((NBUF,)),
        ],
    )
    return f(target.astype(jnp.int32))


# speedup vs baseline: 1.1284x; 1.1284x over previous
import functools

import jax
import jax.numpy as jnp
from jax import lax
from jax.experimental import pallas as pl
from jax.experimental.pallas import tpu as pltpu
from jax.experimental.pallas import tpu_sc as plsc

B = 16384
C = 1000
NC = 2
NS = 16
NW = NC * NS
RPW = B // NW
ZBLK = 512
TBL = 16384


def _zero_body(o_ref):
    o_ref[...] = jnp.zeros_like(o_ref)


_zero_fill = pl.pallas_call(
    _zero_body,
    out_shape=jax.ShapeDtypeStruct((B, C), jnp.float32),
    grid=(B // ZBLK,),
    out_specs=pl.BlockSpec((ZBLK, C), lambda i: (i, 0)),
)


def _ones_body(tgt_hbm, out_hbm, idx_v, table_v, drain_v, sem):
    cid = lax.axis_index("c")
    sid = lax.axis_index("s")
    wid = sid * NC + cid
    base = pl.multiple_of(wid * RPW, 8)

    lanes = lax.iota(jnp.int32, 16)
    for r in range(8):
        table_v[pl.ds(1024 + 2048 * r, 16)] = \
            jnp.where(lanes == r, 1.0, 0.0)

    pltpu.sync_copy(tgt_hbm.at[pl.ds(base, RPW)], idx_v)

    @pl.loop(0, RPW // 16)
    def _grp(g):
        c16 = idx_v[pl.ds(pl.multiple_of(g * 16, 16), 16)]
        o16 = 1024 + jnp.bitwise_and(c16, 7) * 2048
        cb16 = jnp.bitwise_and(c16, ~7)
        for k in range(16):
            o = pl.multiple_of(o16[k], 8)
            cb = pl.multiple_of(cb16[k], 8)
            row = base + g * 16 + k
            pltpu.async_copy(table_v.at[pl.ds(o, 8)],
                             out_hbm.at[row, pl.ds(cb, 8)], sem)

    pltpu.make_async_copy(tgt_hbm.at[pl.ds(0, 4096)], drain_v, sem).wait()


_sc_ones = pl.kernel(
    _ones_body,
    out_type=(),
    mesh=plsc.VectorSubcoreMesh(core_axis_name="c", subcore_axis_name="s"),
    compiler_params=pltpu.CompilerParams(needs_layout_passes=False),
    scratch_types=[
        pltpu.VMEM((RPW,), jnp.int32),
        pltpu.VMEM((TBL,), jnp.float32),
        pltpu.VMEM((4096,), jnp.int32),
        pltpu.SemaphoreType.DMA,
    ],
)


def kernel(target):
    out = jax.new_ref(_zero_fill())
    _sc_ones(target.astype(jnp.int32), out)
    return out[...]
